# baseline (speedup 1.0000x reference)
import jax
import jax.numpy as jnp
from jax.experimental import pallas as pl
from jax.experimental.pallas import tpu as pltpu

_LEVELS = (8, 8, 8, 5, 5, 5)
_BASIS = (1.0, 8.0, 64.0, 512.0, 2560.0, 12800.0)
_D = len(_LEVELS)
_B, _S = 32, 1024
_RNE = 1.5 * 2.0**23


def _fsq_body(z_hbm, q_hbm, idx_hbm, li_hbm, zv, qvp, livp, iv, insem, outsem):
    pltpu.make_async_copy(z_hbm, zv, insem).start()
    pltpu.make_async_copy(z_hbm, zv, insem).wait()
    pend = {}
    acc = jnp.zeros((_B, _S), jnp.float32)
    for j in range(_D):
        s = j % 2
        if j >= 2:
            for c in pend.pop(j - 2):
                c.wait()
        x = zv[j]
        act = (jnp.tanh(x) + 1.0) * 0.5
        y = act * jnp.float32(_LEVELS[j] - 1)
        lif = (y + _RNE) - _RNE
        qvp[s] = (lif / jnp.float32(_LEVELS[j] - 1)) * 2.0 - 1.0
        livp[s] = lif.astype(jnp.int32)
        acc = acc + lif * jnp.float32(_BASIS[j])
        cs = [
            pltpu.make_async_copy(qvp.at[s], q_hbm.at[j], outsem.at[s]),
            pltpu.make_async_copy(livp.at[s], li_hbm.at[j], outsem.at[s]),
        ]
        for c in cs:
            c.start()
        pend[j] = cs
    iv[...] = acc.astype(jnp.int32)
    ic = pltpu.make_async_copy(iv, idx_hbm, insem)
    ic.start()
    for j in sorted(pend):
        for c in pend[j]:
            c.wait()
    ic.wait()


_fsq_tc = pl.pallas_call(
    _fsq_body,
    in_specs=[pl.BlockSpec(memory_space=pl.ANY)],
    out_specs=[
        pl.BlockSpec(memory_space=pl.ANY),
        pl.BlockSpec(memory_space=pl.ANY),
        pl.BlockSpec(memory_space=pl.ANY),
    ],
    out_shape=[
        jax.ShapeDtypeStruct((_D, _B, _S), jnp.float32),
        jax.ShapeDtypeStruct((_B, _S), jnp.int32),
        jax.ShapeDtypeStruct((_D, _B, _S), jnp.int32),
    ],
    scratch_shapes=[
        pltpu.VMEM((_D, _B, _S), jnp.float32),
        pltpu.VMEM((2, _B, _S), jnp.float32),
        pltpu.VMEM((2, _B, _S), jnp.int32),
        pltpu.VMEM((_B, _S), jnp.int32),
        pltpu.SemaphoreType.DMA,
        pltpu.SemaphoreType.DMA((2,)),
    ],
)


def kernel(z):
    q, idx, li = _fsq_tc(z.transpose(2, 0, 1))
    return q.transpose(1, 2, 0), idx, li.transpose(1, 2, 0)


# speedup vs baseline: 1.2720x; 1.2720x over previous
import jax
import jax.numpy as jnp
from jax.experimental import pallas as pl
from jax.experimental.pallas import tpu as pltpu

_LEVELS = (8, 8, 8, 5, 5, 5)
_BASIS = (1.0, 8.0, 64.0, 512.0, 2560.0, 12800.0)
_D = len(_LEVELS)
_B, _S = 32, 1024
_RNE = 1.5 * 2.0**23


def _fsq_tc_body(z_hbm, q_ref, idx_ref, li_ref, z_ref, insem):
    pltpu.make_async_copy(z_hbm, z_ref, insem).start()
    pltpu.make_async_copy(z_hbm, z_ref, insem).wait()
    acc = jnp.zeros((_B, _S), jnp.float32)
    for j in range(_D):
        x = z_ref[j]
        act = (jnp.tanh(x) + 1.0) * 0.5
        y = act * jnp.float32(_LEVELS[j] - 1)
        lif = (y + _RNE) - _RNE
        q_ref[j] = (lif / jnp.float32(_LEVELS[j] - 1)) * 2.0 - 1.0
        li_ref[j] = lif.astype(jnp.int32)
        acc = acc + lif * jnp.float32(_BASIS[j])
    idx_ref[...] = acc.astype(jnp.int32)


_fsq_tc = pl.pallas_call(
    _fsq_tc_body,
    in_specs=[pl.BlockSpec(memory_space=pl.ANY)],
    scratch_shapes=[
        pltpu.VMEM((_D, _B, _S), jnp.float32),
        pltpu.SemaphoreType.DMA,
    ],
    out_shape=[
        jax.ShapeDtypeStruct((_D, _B, _S), jnp.float32),
        jax.ShapeDtypeStruct((_B, _S), jnp.int32),
        jax.ShapeDtypeStruct((_D, _B, _S), jnp.int32),
    ],
)


def kernel(z):
    q, idx, li = _fsq_tc(z.transpose(2, 0, 1))
    return q.transpose(1, 2, 0), idx, li.transpose(1, 2, 0)
